# baseline (speedup 1.0000x reference)
import jax
import jax.numpy as jnp
from jax import lax
from jax.experimental import pallas as pl
from jax.experimental.pallas import tpu as pltpu
from jax.experimental.pallas import tpu_sc as plsc

_NC = 2
_NS = 16
_NW = _NC * _NS

_GRP = 16
_FIRE = 13
_NB_CHUNK = 16


def _repack_body(s0, s1, s2, s3, out_ref):
    out_ref[...] = jnp.concatenate([s0[...], s1[...], s2[...], s3[...]], axis=1)


def _repack_table(table):
    n_rows, d = table.shape
    pack = 128 // d
    q_rows = n_rows // pack
    bbt = 2000
    nblk = q_rows // bbt

    def mk(q):
        return pl.BlockSpec((bbt, d), lambda i, q=q: (i + q * nblk, 0))

    return pl.pallas_call(
        _repack_body,
        grid=(nblk,),
        in_specs=[mk(0), mk(1), mk(2), mk(3)],
        out_specs=pl.BlockSpec((bbt, 128), lambda i: (i, 0)),
        out_shape=jax.ShapeDtypeStruct((q_rows, 128), jnp.float32),
    )(table, table, table, table)


def _make_sc_gather(B: int, n_cat: int, d: int, q_rows: int):
    rows_per_b = n_cat
    chunk = _NB_CHUNK * rows_per_b
    b_per_w = B // _NW
    n_chunks = b_per_w // _NB_CHUNK
    n_grp = chunk // _GRP
    dc = n_cat * d
    half = d // 2

    def body(t128_hbm, gidx_hbm, qd_hbm, out_hbm, gv_v, qd_v, rows_v, sel_v, sem):
        wid = lax.axis_index("s") * _NC + lax.axis_index("c")
        b_base = wid * b_per_w

        def chunk_body(c, _):
            b0 = b_base + c * _NB_CHUNK
            off = pl.multiple_of(b0 * rows_per_b, chunk)
            pltpu.sync_copy(gidx_hbm.at[pl.ds(off, chunk)], gv_v)
            pltpu.sync_copy(qd_hbm.at[pl.ds(off, chunk)], qd_v)
            for g0 in range(0, n_grp, _FIRE):
                cps = []
                for g in range(g0, min(g0 + _FIRE, n_grp)):
                    gv = gv_v[pl.ds(g * _GRP, _GRP)]
                    cps.append(
                        pltpu.async_copy(
                            t128_hbm.at[gv],
                            rows_v.at[pl.ds(g * _GRP, _GRP)],
                            sem,
                        )
                    )
                for cp in cps:
                    cp.wait()

            def sel_body(g, carry):
                row, lane = carry
                qoff = qd_v[pl.ds(g * _GRP, _GRP)]
                for i in range(_GRP):
                    j = g * _GRP + i
                    q = pl.multiple_of(qoff[i], half)
                    la = pl.multiple_of(lane, half)
                    sel_v[row, pl.ds(la, half)] = rows_v[j, pl.ds(q, half)]
                    sel_v[row, pl.ds(pl.multiple_of(la + half, half), half)] = (
                        rows_v[j, pl.ds(pl.multiple_of(q + half, half), half)]
                    )
                    wrap = lane + d == dc
                    row = jnp.where(wrap, row + 1, row)
                    lane = jnp.where(wrap, 0, lane + d)
                return row, lane

            lax.fori_loop(0, n_grp, sel_body, (0, 0))
            pltpu.sync_copy(sel_v, out_hbm.at[pl.ds(b0, _NB_CHUNK)])
            return 0

        lax.fori_loop(0, n_chunks, chunk_body, 0)

    mesh = plsc.VectorSubcoreMesh(
        core_axis_name="c", subcore_axis_name="s", num_cores=_NC, num_subcores=_NS
    )
    return pl.kernel(
        body,
        out_type=jax.ShapeDtypeStruct((B, dc), jnp.float32),
        mesh=mesh,
        scratch_types=[
            pltpu.VMEM((chunk,), jnp.int32),
            pltpu.VMEM((chunk,), jnp.int32),
            pltpu.VMEM((chunk, 128), jnp.float32),
            pltpu.VMEM((_NB_CHUNK, dc), jnp.float32),
            pltpu.SemaphoreType.DMA,
        ],
    )


def _tc_body(xn_ref, w_ref, nb_ref, cat_ref, cb_ref, out_ref):
    bb = out_ref.shape[0]
    n_num, d_emb = nb_ref.shape[1], nb_ref.shape[2]
    n_cat = cb_ref.shape[1]
    num2 = jnp.dot(
        xn_ref[...],
        w_ref[...],
        preferred_element_type=jnp.float32,
        precision=jax.lax.Precision.HIGHEST,
    )
    num3 = num2.reshape(bb, n_num, d_emb) + nb_ref[...]
    cat3 = cat_ref[...].reshape(bb, n_cat, d_emb) + cb_ref[...]
    out_ref[...] = jnp.concatenate([num3, cat3], axis=1)


def kernel(x_num, x_cat, num_w, num_b, cat_table, cat_bias):
    B, n_num, n_bins = x_num.shape
    n_cat = x_cat.shape[1]
    d_emb = cat_table.shape[1]
    card = cat_table.shape[0] // n_cat

    t128 = _repack_table(cat_table)
    q_rows = t128.shape[0]

    offsets = (jnp.arange(n_cat, dtype=jnp.int32) * card)[None]
    idx = (x_cat.astype(jnp.int32) + offsets).reshape(-1)
    q = idx // q_rows
    gidx = idx - q * q_rows
    qd = q * d_emb
    gather = _make_sc_gather(B, n_cat, d_emb, q_rows)
    cat2 = gather(t128, gidx, qd)

    dn = n_num * n_bins
    eye = jnp.eye(n_num, dtype=jnp.float32)
    w_blk = (eye[:, None, :, None] * num_w[:, :, None, :]).reshape(
        dn, n_num * d_emb
    )
    bb = 256
    out = pl.pallas_call(
        _tc_body,
        grid=(B // bb,),
        in_specs=[
            pl.BlockSpec((bb, dn), lambda i: (i, 0)),
            pl.BlockSpec((dn, n_num * d_emb), lambda i: (0, 0)),
            pl.BlockSpec((1, n_num, d_emb), lambda i: (0, 0, 0)),
            pl.BlockSpec((bb, n_cat * d_emb), lambda i: (i, 0)),
            pl.BlockSpec((1, n_cat, d_emb), lambda i: (0, 0, 0)),
        ],
        out_specs=pl.BlockSpec((bb, n_num + n_cat, d_emb), lambda i: (i, 0, 0)),
        out_shape=jax.ShapeDtypeStruct((B, n_num + n_cat, d_emb), jnp.float32),
    )(
        x_num.reshape(B, dn),
        w_blk,
        num_b.reshape(1, n_num, d_emb),
        cat2,
        cat_bias.reshape(1, n_cat, d_emb),
    )
    return out
